# baseline (speedup 1.0000x reference)
import functools

import jax
import jax.numpy as jnp
from jax import lax
from jax.experimental import pallas as pl
from jax.experimental.pallas import tpu as pltpu
from jax.experimental.pallas import tpu_sc as plsc

_NUM_FIELDS = 26
_VOCAB = 100000
_EMBED_DIM = 32
_BATCH = 16384
_NUM_NUMERICAL = 13

_PACK = 128 // _EMBED_DIM
_NC = 2
_NS = 16
_NW = _NC * _NS
_TOTAL = _BATCH * _NUM_FIELDS
_PER_W = _TOTAL // _NW
_CHUNK = 128
_N_CHUNKS = _PER_W // _CHUNK

_mesh = plsc.VectorSubcoreMesh(core_axis_name="c", subcore_axis_name="s")


@functools.partial(
    pl.kernel,
    out_type=jax.ShapeDtypeStruct((_TOTAL, 128), jnp.float32),
    mesh=_mesh,
    scratch_types=[
        pltpu.VMEM((_PER_W,), jnp.int32),
        pltpu.VMEM((_CHUNK, 128), jnp.float32),
        pltpu.SemaphoreType.DMA,
    ],
)
def _gather_kernel(idx_hbm, tab_hbm, out_hbm, idx_v, rows_v, sem):
    wid = lax.axis_index("s") * _NC + lax.axis_index("c")
    base = pl.multiple_of(wid * _PER_W, _CHUNK)
    pltpu.sync_copy(idx_hbm.at[pl.ds(base, _PER_W)], idx_v)

    @pl.loop(0, _N_CHUNKS)
    def _chunk(c):
        off = pl.multiple_of(c * _CHUNK, _CHUNK)
        pltpu.async_copy(
            tab_hbm.at[idx_v.at[pl.ds(off, _CHUNK)]], rows_v, sem
        ).wait()
        pltpu.sync_copy(rows_v, out_hbm.at[pl.ds(base + off, _CHUNK)])


_XB = 1024


def _extract_body(off_ref, wide_ref, out_ref):
    off = off_ref[:, :]
    acc = jnp.zeros((_XB, _EMBED_DIM), jnp.float32)
    for j in range(_PACK):
        m = (off == j).astype(jnp.float32)
        acc = acc + wide_ref[:, 32 * j:32 * (j + 1)] * m
    out_ref[:, :] = acc


def _extract(wide, off):
    grid = _TOTAL // _XB
    return pl.pallas_call(
        _extract_body,
        grid=(grid,),
        in_specs=[
            pl.BlockSpec((_XB, 1), lambda i: (i, 0)),
            pl.BlockSpec((_XB, 128), lambda i: (i, 0)),
        ],
        out_specs=pl.BlockSpec((_XB, _EMBED_DIM), lambda i: (i, 0)),
        out_shape=jax.ShapeDtypeStruct((_TOTAL, _EMBED_DIM), jnp.float32),
    )(off, wide)


def kernel(numerical, cat_indices, tables):
    flat = (
        cat_indices.astype(jnp.int32)
        + jnp.arange(_NUM_FIELDS, dtype=jnp.int32) * _VOCAB
    ).reshape(-1)
    tab = tables.reshape(_NUM_FIELDS * _VOCAB // _PACK, 128)
    wide = _gather_kernel(flat // _PACK, tab)
    rows = _extract(wide, (flat % _PACK).reshape(-1, 1))
    emb = rows.reshape(_BATCH, _NUM_FIELDS * _EMBED_DIM)
    return jnp.concatenate([numerical, emb], axis=1)
